# baseline (speedup 1.0000x reference)
import jax
import jax.numpy as jnp
from jax.experimental import pallas as pl
from jax.experimental.pallas import tpu as pltpu

D = 4096
EPS = 1e-5
ROWS_PER_BLOCK = 512


DCHUNK = 1024


def _stats_body(x_ref, m_ref, sum_ref, sq_ref, cnt_ref):
    @pl.when(pl.program_id(0) == 0)
    def _init():
        sum_ref[...] = jnp.zeros_like(sum_ref)
        sq_ref[...] = jnp.zeros_like(sq_ref)
        cnt_ref[...] = jnp.zeros_like(cnt_ref)

    m = m_ref[...]
    cnt_ref[...] += jnp.sum(m, axis=0, keepdims=True)
    R = x_ref.shape[0]
    for d in range(D // DCHUNK):
        def body(g, acc):
            s, q = acc
            xs = x_ref[pl.ds(g * 8, 8), d * DCHUNK:(d + 1) * DCHUNK]
            ms = m_ref[pl.ds(g * 8, 8), :]
            xm = xs * ms
            return (s + xm, q + xm * xs)

        z = jnp.zeros((8, DCHUNK), jnp.float32)
        s, q = jax.lax.fori_loop(0, R // 8, body, (z, z))
        sum_ref[0:1, d * DCHUNK:(d + 1) * DCHUNK] += jnp.sum(s, axis=0, keepdims=True)
        sq_ref[0:1, d * DCHUNK:(d + 1) * DCHUNK] += jnp.sum(q, axis=0, keepdims=True)


def _norm_body(x_ref, m_ref, sum_ref, sq_ref, cnt_ref, g_ref, b_ref, o_ref):
    n = jnp.maximum(cnt_ref[0, 0], 1.0)
    rn = 1.0 / n
    mean = sum_ref[...] * rn
    var = jnp.maximum(sq_ref[...] * rn - mean * mean, 0.0)
    inv = jax.lax.rsqrt(var + EPS)
    scale = inv * g_ref[...]
    shift = b_ref[...] - mean * scale
    x = x_ref[...]
    xn = x * scale + shift
    o_ref[...] = jnp.where(m_ref[...] > 0.0, xn, x)


def kernel(x, mask, gamma, beta):
    B, T, _D = x.shape
    N = B * T
    xf = x.reshape(N, D)
    mf = mask.reshape(N, 1).astype(jnp.float32)
    g2 = gamma.reshape(1, D)
    b2 = beta.reshape(1, D)

    R = ROWS_PER_BLOCK
    nblk = N // R

    sums, sqs, cnt = pl.pallas_call(
        _stats_body,
        grid=(nblk,),
        in_specs=[
            pl.BlockSpec((R, D), lambda i: (i, 0)),
            pl.BlockSpec((R, 1), lambda i: (i, 0)),
        ],
        out_specs=[
            pl.BlockSpec((1, D), lambda i: (0, 0)),
            pl.BlockSpec((1, D), lambda i: (0, 0)),
            pl.BlockSpec((1, 1), lambda i: (0, 0)),
        ],
        out_shape=[
            jax.ShapeDtypeStruct((1, D), jnp.float32),
            jax.ShapeDtypeStruct((1, D), jnp.float32),
            jax.ShapeDtypeStruct((1, 1), jnp.float32),
        ],
    )(xf, mf)

    out = pl.pallas_call(
        _norm_body,
        grid=(nblk,),
        in_specs=[
            pl.BlockSpec((R, D), lambda i: (i, 0)),
            pl.BlockSpec((R, 1), lambda i: (i, 0)),
            pl.BlockSpec((1, D), lambda i: (0, 0)),
            pl.BlockSpec((1, D), lambda i: (0, 0)),
            pl.BlockSpec((1, 1), lambda i: (0, 0)),
            pl.BlockSpec((1, D), lambda i: (0, 0)),
            pl.BlockSpec((1, D), lambda i: (0, 0)),
        ],
        out_specs=pl.BlockSpec((R, D), lambda i: (i, 0)),
        out_shape=jax.ShapeDtypeStruct((N, D), jnp.float32),
    )(xf, mf, sums, sqs, cnt, g2, b2)

    return out.reshape(B, T, D)


# speedup vs baseline: 3.1522x; 3.1522x over previous
import jax
import jax.numpy as jnp
from jax.experimental import pallas as pl
from jax.experimental.pallas import tpu as pltpu

D = 4096
EPS = 1e-5
ROWS_PER_BLOCK = 512


def _body(x_ref, m_ref, mt_ref, g_ref, b_ref, o_ref, sum_ref, sq_ref, cnt_ref):
    p = pl.program_id(0)
    i = pl.program_id(1)

    @pl.when(p == 0)
    def _stats():
        @pl.when(i == 0)
        def _init():
            sum_ref[...] = jnp.zeros_like(sum_ref)
            sq_ref[...] = jnp.zeros_like(sq_ref)
            cnt_ref[...] = jnp.zeros_like(cnt_ref)

        x = x_ref[...]
        mt = mt_ref[...]
        sum_ref[...] += jax.lax.dot_general(
            mt, x, (((1,), (0,)), ((), ())), preferred_element_type=jnp.float32)
        sq_ref[...] += jax.lax.dot_general(
            mt, x * x, (((1,), (0,)), ((), ())), preferred_element_type=jnp.float32)
        cnt_ref[...] += jnp.sum(mt, axis=1, keepdims=True)

    @pl.when(p == 1)
    def _normalize():
        n = jnp.maximum(cnt_ref[0, 0], 1.0)
        rn = 1.0 / n
        mean = sum_ref[...] * rn
        var = jnp.maximum(sq_ref[...] * rn - mean * mean, 0.0)
        inv = jax.lax.rsqrt(var + EPS)
        scale = inv * g_ref[...]
        shift = b_ref[...] - mean * scale
        x = x_ref[...]
        xn = x * scale + shift
        o_ref[...] = jnp.where(m_ref[...] > 0.0, xn, x)


def kernel(x, mask, gamma, beta):
    B, T, _D = x.shape
    N = B * T
    xf = x.reshape(N, D)
    mf = mask.reshape(N, 1).astype(jnp.float32)
    mft = mask.reshape(1, N).astype(jnp.float32)
    g2 = gamma.reshape(1, D)
    b2 = beta.reshape(1, D)

    R = ROWS_PER_BLOCK
    nblk = N // R

    out = pl.pallas_call(
        _body,
        grid=(2, nblk),
        in_specs=[
            pl.BlockSpec((R, D), lambda p, i: (i, 0)),
            pl.BlockSpec((R, 1), lambda p, i: (i, 0)),
            pl.BlockSpec((1, R), lambda p, i: (0, i)),
            pl.BlockSpec((1, D), lambda p, i: (0, 0)),
            pl.BlockSpec((1, D), lambda p, i: (0, 0)),
        ],
        out_specs=pl.BlockSpec((R, D), lambda p, i: (jnp.where(p == 0, 0, i), 0)),
        out_shape=jax.ShapeDtypeStruct((N, D), jnp.float32),
        scratch_shapes=[
            pltpu.VMEM((1, D), jnp.float32),
            pltpu.VMEM((1, D), jnp.float32),
            pltpu.VMEM((1, 1), jnp.float32),
        ],
    )(xf, mf, mft, g2, b2)

    return out.reshape(B, T, D)
